# baseline (speedup 1.0000x reference)
import functools

import jax
import jax.numpy as jnp
from jax import lax
from jax.experimental import pallas as pl
from jax.experimental.pallas import tpu as pltpu
from jax.experimental.pallas import tpu_sc as plsc

B, T = 1024, 50
NUM = 16
N_CAT = 26
VOCAB = 100000
EDIM = 32
OUT_F = NUM + N_CAT * EDIM
TP = 56
IP = 64


def _make_sc_gather(num_workers: int):
    b_per_w = B // num_workers
    mesh = plsc.VectorSubcoreMesh(core_axis_name="c", subcore_axis_name="s")

    @functools.partial(
        pl.kernel,
        mesh=mesh,
        compiler_params=pltpu.CompilerParams(use_tc_tiling_on_sc=False),
        out_type=jax.ShapeDtypeStruct((B, N_CAT, TP, EDIM), jnp.float32),
        scratch_types=[
            pltpu.VMEM((N_CAT * IP,), jnp.int32),
            pltpu.VMEM((N_CAT, TP, EDIM), jnp.float32),
            pltpu.SemaphoreType.DMA,
        ],
    )
    def k(cat_hbm, tab_hbm, x_hbm, idx_v, vbuf, sem):
        nc = plsc.get_sparse_core_info().num_cores
        wid = lax.axis_index("s") * nc + lax.axis_index("c")

        def body_b(bi, carry):
            b = wid * b_per_w + bi
            pltpu.sync_copy(cat_hbm.at[pl.ds(b * (N_CAT * IP), N_CAT * IP)], idx_v)

            def body_c(c, cc):
                pltpu.async_copy(
                    tab_hbm.at[idx_v.at[pl.ds(c * IP, TP)]],
                    vbuf.at[c],
                    sem,
                ).wait()
                return cc

            lax.fori_loop(0, N_CAT, body_c, 0)
            pltpu.sync_copy(vbuf, x_hbm.at[b])
            return carry

        lax.fori_loop(0, b_per_w, body_b, 0)

    return k


_GB = 8


def _tc_body(x_ref, num_ref, out_ref):
    x = x_ref[...]
    xt = jnp.swapaxes(x, 2, 3)[:, :, :, :T]
    for g in range(_GB):
        out_ref[g, 0:NUM, :] = num_ref[g]
        out_ref[g, NUM:, :] = xt[g].reshape(N_CAT * EDIM, T)


def kernel(numerical, categorical, tables):
    info = plsc.get_sparse_core_info()
    nw = info.num_cores * info.num_subcores
    tab_flat = tables.reshape(N_CAT * VOCAB, EDIM)
    offs = (jnp.arange(N_CAT, dtype=jnp.int32) * VOCAB)[None, :, None]
    cat_flat = jnp.pad(categorical + offs, ((0, 0), (0, 0), (0, IP - T))).reshape(-1)
    x = _make_sc_gather(nw)(cat_flat, tab_flat)
    out = pl.pallas_call(
        _tc_body,
        grid=(B // _GB,),
        in_specs=[
            pl.BlockSpec((_GB, N_CAT, TP, EDIM), lambda i: (i, 0, 0, 0)),
            pl.BlockSpec((_GB, NUM, T), lambda i: (i, 0, 0)),
        ],
        out_specs=pl.BlockSpec((_GB, OUT_F, T), lambda i: (i, 0, 0)),
        out_shape=jax.ShapeDtypeStruct((B, OUT_F, T), jnp.float32),
    )(x, numerical)
    return out
